# baseline (speedup 1.0000x reference)
import jax
import jax.numpy as jnp
from jax.experimental import pallas as pl
from jax.experimental.pallas import tpu as pltpu

_ALPHA = 0.25
_GAMMA = 2.0

_ROWS = 16384
_COLS = 2048
_BLOCK_ROWS = 256


def _focal_body(x_ref, t_ref, loss_ref, cnt_ref):
    i = pl.program_id(0)

    @pl.when(i == 0)
    def _init():
        loss_ref[0, 0] = 0.0
        cnt_ref[0, 0] = 0

    x = x_ref[...]
    t = t_ref[...]

    e = jnp.exp(-jnp.abs(x))
    inv = 1.0 / (1.0 + e)
    p = jnp.where(x >= 0.0, inv, e * inv)
    ce = jnp.maximum(x, 0.0) - x * t + jnp.log1p(e)
    p_t = p * t + (1.0 - p) * (1.0 - t)
    alpha_t = _ALPHA * t + (1.0 - _ALPHA) * (1.0 - t)
    one_m = 1.0 - p_t
    loss = alpha_t * ce * (one_m * one_m)

    union = (x != 0.0) | (t != 0.0)
    loss = jnp.where(union, loss, 0.0)

    loss_ref[0, 0] += jnp.sum(loss)
    cnt_ref[0, 0] += jnp.sum((t > 0.5).astype(jnp.int32))

    @pl.when(i == pl.num_programs(0) - 1)
    def _finish():
        total = loss_ref[0, 0]
        num_pos = jnp.maximum(cnt_ref[0, 0], 1).astype(jnp.float32)
        loss_ref[0, 0] = total / num_pos


def kernel(predicted_foreground_masks, peak_normalized_images):
    x = predicted_foreground_masks.reshape(_ROWS, _COLS)
    t = peak_normalized_images.reshape(_ROWS, _COLS)
    grid = _ROWS // _BLOCK_ROWS

    loss, _cnt = pl.pallas_call(
        _focal_body,
        grid=(grid,),
        in_specs=[
            pl.BlockSpec((_BLOCK_ROWS, _COLS), lambda i: (i, 0)),
            pl.BlockSpec((_BLOCK_ROWS, _COLS), lambda i: (i, 0)),
        ],
        out_specs=[
            pl.BlockSpec((1, 1), lambda i: (0, 0), memory_space=pltpu.SMEM),
            pl.BlockSpec((1, 1), lambda i: (0, 0), memory_space=pltpu.SMEM),
        ],
        out_shape=[
            jax.ShapeDtypeStruct((1, 1), jnp.float32),
            jax.ShapeDtypeStruct((1, 1), jnp.int32),
        ],
    )(x, t)
    return loss[0, 0]
